# baseline (speedup 1.0000x reference)
import functools

import jax
import jax.numpy as jnp
from jax import lax
from jax.experimental import pallas as pl
from jax.experimental.pallas import tpu as pltpu

DIM = 768
NUM_EXPERTS = 64
N_TOKENS = 32768
TOPK = 512
BT = 2048


def _gate_body(x_ref, w_ref, b_ref, out_ref):
    xb = x_ref[...]
    w = w_ref[...]
    logits = lax.dot_general(
        xb, w, (((1,), (1,)), ((), ())),
        preferred_element_type=jnp.float32)
    logits = logits + b_ref[...][None, :]
    out_ref[...] = jax.nn.sigmoid(logits)


@functools.partial(jax.jit, static_argnames=())
def _gate_scores(x, W_gate, b_gate):
    grid = (N_TOKENS // BT,)
    return pl.pallas_call(
        _gate_body,
        grid=grid,
        in_specs=[
            pl.BlockSpec((BT, DIM), lambda i: (i, 0)),
            pl.BlockSpec((NUM_EXPERTS, DIM), lambda i: (0, 0)),
            pl.BlockSpec((NUM_EXPERTS,), lambda i: (0,)),
        ],
        out_specs=pl.BlockSpec((BT, NUM_EXPERTS), lambda i: (i, 0)),
        out_shape=jax.ShapeDtypeStruct((N_TOKENS, NUM_EXPERTS), jnp.float32),
    )(x, W_gate, b_gate)


def kernel(x, W_gate, b_gate):
    scores = _gate_scores(x, W_gate, b_gate)
    scores_t = scores.T
    top_scores, idx = lax.top_k(scores_t, TOPK)
    return top_scores, idx


# speedup vs baseline: 4.0428x; 4.0428x over previous
import functools

import jax
import jax.numpy as jnp
from jax import lax
from jax.experimental import pallas as pl
from jax.experimental.pallas import tpu as pltpu
from jax.experimental.pallas import tpu_sc as plsc

DIM = 768
NUM_EXPERTS = 64
N_TOKENS = 32768
TOPK = 512
BT = 2048

L = 16
NV = N_TOKENS // L
GT_PAD = 528
TIE_PAD = 544


def _gate_body(x_ref, w_ref, b_ref, out_ref):
    xb = x_ref[...]
    w = w_ref[...]
    logits = lax.dot_general(
        w, xb, (((1,), (1,)), ((), ())),
        preferred_element_type=jnp.float32)
    logits = logits + b_ref[...][:, None]
    scores = jax.nn.sigmoid(logits)
    out_ref[...] = lax.bitcast_convert_type(scores, jnp.int32)


def _gate_scores(x, W_gate, b_gate):
    grid = (N_TOKENS // BT,)
    return pl.pallas_call(
        _gate_body,
        grid=grid,
        in_specs=[
            pl.BlockSpec((BT, DIM), lambda i: (i, 0)),
            pl.BlockSpec((NUM_EXPERTS, DIM), lambda i: (0, 0)),
            pl.BlockSpec((NUM_EXPERTS,), lambda i: (0,)),
        ],
        out_specs=pl.BlockSpec((NUM_EXPERTS, BT), lambda i: (0, i)),
        out_shape=jax.ShapeDtypeStruct((NUM_EXPERTS, N_TOKENS), jnp.int32),
    )(x, W_gate, b_gate)


def _iota16():
    return lax.broadcasted_iota(jnp.int32, (L,), 0)


def _find_threshold(hist, nvreg, target, smem, slot):
    iota = _iota16()

    def cond(state):
        _, carry = state
        return carry < target

    def body(state):
        j, carry = state
        v = hist[pl.ds(j * L, L)]
        s = jnp.sum(v)
        new = carry + s

        @pl.when(new >= target)
        def _():
            rv = lax.rev(v, (0,))
            dcum = lax.rev(plsc.cumsum(rv), (0,)) + carry
            cond_v = (dcum >= target).astype(jnp.int32)
            m = jnp.sum(cond_v)
            lane = m - 1
            sel = iota == lane
            zeros = jnp.zeros((L,), jnp.int32)
            cgt = jnp.sum(jnp.where(sel, dcum - v, zeros))
            smem[slot] = j * L + lane
            smem[slot + 1] = cgt

        return j - 1, new

    lax.while_loop(cond, body, (jnp.int32(nvreg - 1), jnp.int32(0)))


def _topk_row(scores_hbm, vals_hbm, idx_hbm, e,
              keys, hist, gtk, gti, gtk2, gti2, tie, bins, outv, outi, smem):
    iota = _iota16()
    zeros = jnp.zeros((L,), jnp.int32)
    ones = jnp.ones((L,), jnp.int32)

    pltpu.sync_copy(scores_hbm.at[pl.ds(e * N_TOKENS, N_TOKENS)], keys)

    U = 4

    def clear_body(i, _):
        for u in range(U):
            hist[pl.ds((i * U + u) * L, L)] = zeros
        return 0

    lax.fori_loop(0, NV // U, clear_body, 0)

    def hist_hi_body(i, _):
        for u in range(U):
            k = keys[pl.ds((i * U + u) * L, L)]
            plsc.addupdate_scatter(hist, [k >> 15], ones)
        return 0

    lax.fori_loop(0, NV // U, hist_hi_body, 0)

    _find_threshold(hist, NV, jnp.int32(TOPK), smem, 0)
    h_star = smem[0]
    c_gt = smem[1]

    lax.fori_loop(0, NV // U, clear_body, 0)

    def hist_lo_body(i, _):
        for u in range(U):
            k = keys[pl.ds((i * U + u) * L, L)]
            eligible = (k >> 15) == h_star
            plsc.addupdate_scatter(hist, [k & 0x7FFF], ones, mask=eligible)
        return 0

    lax.fori_loop(0, NV // U, hist_lo_body, 0)

    _find_threshold(hist, NV, TOPK - c_gt, smem, 2)
    l_star = smem[2]
    c_gt2 = smem[3]

    t_key = (h_star << 15) | l_star
    c_sel = c_gt + c_gt2

    def collect_body(i, carry):
        off_gt, off_tie = carry
        for u in range(U):
            k = keys[pl.ds((i * U + u) * L, L)]
            idxv = iota + (i * U + u) * L
            gt = k > t_key
            tie_ok = jnp.logical_and(k == t_key, off_tie < TIE_PAD - L)
            plsc.store_compressed(gtk.at[pl.ds(off_gt, L)], k, mask=gt)
            plsc.store_compressed(gti.at[pl.ds(off_gt, L)], idxv, mask=gt)
            plsc.store_compressed(tie.at[pl.ds(off_tie, L)], idxv, mask=tie_ok)
            off_gt = off_gt + jnp.sum(gt.astype(jnp.int32))
            off_tie = off_tie + jnp.sum(tie_ok.astype(jnp.int32))
        return off_gt, off_tie

    lax.fori_loop(0, NV // U, collect_body, (jnp.int32(0), jnp.int32(0)))

    nv_sel = GT_PAD // L
    bufs = [(gtk, gti), (gtk2, gti2)]
    for p in range(6):
        srck, srci = bufs[p % 2]
        dstk, dsti = bufs[(p + 1) % 2]
        shift = 5 * p

        bins[pl.ds(0, L)] = zeros
        bins[pl.ds(L, L)] = zeros

        def count_body(i, _, srck=srck, shift=shift):
            pos = iota + i * L
            valid = pos < c_sel
            k = srck[pl.ds(i * L, L)]
            dd = 31 - ((k >> shift) & 31)
            plsc.addupdate_scatter(bins, [dd], ones, mask=valid)
            return 0

        lax.fori_loop(0, nv_sel, count_body, 0)

        v0 = bins[pl.ds(0, L)]
        v1 = bins[pl.ds(L, L)]
        bins[pl.ds(0, L)] = plsc.cumsum(v0) - v0
        bins[pl.ds(L, L)] = plsc.cumsum(v1) - v1 + jnp.sum(v0)

        def perm_body(i, _, srck=srck, srci=srci, dstk=dstk, dsti=dsti,
                      shift=shift):
            pos = iota + i * L
            valid = pos < c_sel
            k = srck[pl.ds(i * L, L)]
            iv = srci[pl.ds(i * L, L)]
            dd = 31 - ((k >> shift) & 31)
            occ, lm = plsc.scan_count(dd, mask=valid)
            base = plsc.load_gather(bins, [dd])
            dest = base + occ - 1
            plsc.store_scatter(dstk, [dest], k, mask=valid)
            plsc.store_scatter(dsti, [dest], iv, mask=valid)
            plsc.addupdate_scatter(bins, [dd], occ,
                                   mask=jnp.logical_and(lm, valid))
            return 0

        lax.fori_loop(0, nv_sel, perm_body, 0)

    t_vec = zeros + t_key

    def out_body(i, _):
        pos = iota + i * L
        in_gt = pos < c_sel
        sk = gtk[pl.ds(i * L, L)]
        kbits = jnp.where(in_gt, sk, t_vec)
        outv[pl.ds(i * L, L)] = plsc.bitcast(kbits, jnp.float32)
        tpos = jnp.maximum(pos - c_sel, 0)
        tidx = plsc.load_gather(tie, [tpos])
        gidx = gti[pl.ds(i * L, L)]
        outi[pl.ds(i * L, L)] = jnp.where(in_gt, gidx, tidx)
        return 0

    lax.fori_loop(0, TOPK // L, out_body, 0)

    pltpu.sync_copy(outv, vals_hbm.at[pl.ds(e * TOPK, TOPK)])
    pltpu.sync_copy(outi, idx_hbm.at[pl.ds(e * TOPK, TOPK)])


def _make_topk_sc():
    mesh = plsc.VectorSubcoreMesh(core_axis_name="c", subcore_axis_name="s")

    @functools.partial(
        pl.kernel,
        out_type=(
            jax.ShapeDtypeStruct((NUM_EXPERTS * TOPK,), jnp.float32),
            jax.ShapeDtypeStruct((NUM_EXPERTS * TOPK,), jnp.int32),
        ),
        mesh=mesh,
        compiler_params=pltpu.CompilerParams(needs_layout_passes=False),
        scratch_types=[
            pltpu.VMEM((N_TOKENS,), jnp.int32),
            pltpu.VMEM((N_TOKENS,), jnp.int32),
            pltpu.VMEM((GT_PAD,), jnp.int32),
            pltpu.VMEM((GT_PAD,), jnp.int32),
            pltpu.VMEM((GT_PAD,), jnp.int32),
            pltpu.VMEM((GT_PAD,), jnp.int32),
            pltpu.VMEM((TIE_PAD,), jnp.int32),
            pltpu.VMEM((2 * L,), jnp.int32),
            pltpu.VMEM((TOPK,), jnp.float32),
            pltpu.VMEM((TOPK,), jnp.int32),
            pltpu.SMEM((8,), jnp.int32),
        ],
    )
    def topk_sc(scores_hbm, vals_hbm, idx_hbm,
                keys, hist, gtk, gti, gtk2, gti2, tie, bins, outv, outi,
                smem):
        wid = lax.axis_index("s") * 2 + lax.axis_index("c")
        for r in range(2):
            _topk_row(scores_hbm, vals_hbm, idx_hbm, wid * 2 + r,
                      keys, hist, gtk, gti, gtk2, gti2, tie, bins,
                      outv, outi, smem)

    return topk_sc


_topk_sc = _make_topk_sc()


@jax.jit
def kernel(x, W_gate, b_gate):
    score_bits = _gate_scores(x, W_gate, b_gate)
    vals, idx = _topk_sc(score_bits.reshape(-1))
    return vals.reshape(NUM_EXPERTS, TOPK), idx.reshape(NUM_EXPERTS, TOPK)


# speedup vs baseline: 4.2928x; 1.0618x over previous
import functools

import jax
import jax.numpy as jnp
from jax import lax
from jax.experimental import pallas as pl
from jax.experimental.pallas import tpu as pltpu
from jax.experimental.pallas import tpu_sc as plsc

DIM = 768
NUM_EXPERTS = 64
N_TOKENS = 32768
TOPK = 512
BT = 2048

L = 16
NV = N_TOKENS // L
CAP = 1056


def _gate_body(x_ref, w_ref, b_ref, out_ref):
    xb = x_ref[...]
    w = w_ref[...]
    logits = lax.dot_general(
        w, xb, (((1,), (1,)), ((), ())),
        preferred_element_type=jnp.float32)
    logits = logits + b_ref[...][:, None]
    scores = jax.nn.sigmoid(logits)
    out_ref[...] = lax.bitcast_convert_type(scores, jnp.int32)


def _gate_scores(x, W_gate, b_gate):
    grid = (N_TOKENS // BT,)
    return pl.pallas_call(
        _gate_body,
        grid=grid,
        in_specs=[
            pl.BlockSpec((BT, DIM), lambda i: (i, 0)),
            pl.BlockSpec((NUM_EXPERTS, DIM), lambda i: (0, 0)),
            pl.BlockSpec((NUM_EXPERTS,), lambda i: (0,)),
        ],
        out_specs=pl.BlockSpec((NUM_EXPERTS, BT), lambda i: (0, i)),
        out_shape=jax.ShapeDtypeStruct((NUM_EXPERTS, N_TOKENS), jnp.int32),
    )(x, W_gate, b_gate)


def _iota16():
    return lax.broadcasted_iota(jnp.int32, (L,), 0)


def _find_threshold(hist, nvreg, target, smem, slot):
    iota = _iota16()

    def cond(state):
        _, carry = state
        return carry < target

    def body(state):
        j, carry = state
        v = hist[pl.ds(j * L, L)]
        s = jnp.sum(v)
        new = carry + s

        @pl.when(new >= target)
        def _():
            rv = lax.rev(v, (0,))
            dcum = lax.rev(plsc.cumsum(rv), (0,)) + carry
            cond_v = (dcum >= target).astype(jnp.int32)
            m = jnp.sum(cond_v)
            lane = m - 1
            sel = iota == lane
            zeros = jnp.zeros((L,), jnp.int32)
            smem[slot] = j * L + lane
            smem[slot + 1] = jnp.sum(jnp.where(sel, dcum - v, zeros))
            smem[slot + 2] = jnp.sum(jnp.where(sel, v, zeros))

        return j - 1, new

    lax.while_loop(cond, body, (jnp.int32(nvreg - 1), jnp.int32(0)))


def _topk_row(scores_hbm, vals_hbm, idx_hbm, e,
              keys, hist, selk, seli, selk2, seli2, bins, outv, smem):
    iota = _iota16()
    zeros = jnp.zeros((L,), jnp.int32)
    ones = jnp.ones((L,), jnp.int32)

    pltpu.sync_copy(scores_hbm.at[pl.ds(e * N_TOKENS, N_TOKENS)], keys)

    U = 8

    def clear_body(i, _):
        for u in range(U):
            hist[pl.ds((i * U + u) * L, L)] = zeros
        return 0

    lax.fori_loop(0, NV // U, clear_body, 0)

    def hist_hi_body(i, _):
        for u in range(U):
            k = keys[pl.ds((i * U + u) * L, L)]
            plsc.addupdate_scatter(hist, [k >> 15], ones)
        return 0

    lax.fori_loop(0, NV // U, hist_hi_body, 0)

    _find_threshold(hist, NV, jnp.int32(TOPK), smem, 0)
    h_star = smem[0]
    c_gt = smem[1]

    lax.fori_loop(0, NV // U, clear_body, 0)

    def hist_lo_body(i, _):
        for u in range(U):
            k = keys[pl.ds((i * U + u) * L, L)]
            eligible = (k >> 15) == h_star
            plsc.addupdate_scatter(hist, [k & 0x7FFF], ones, mask=eligible)
        return 0

    lax.fori_loop(0, NV // U, hist_lo_body, 0)

    _find_threshold(hist, NV, TOPK - c_gt, smem, 3)
    l_star = smem[3]
    c_gt2 = smem[4]
    cnt_at = smem[5]

    t_key = (h_star << 15) | l_star
    c_sel = c_gt + c_gt2
    m = jnp.minimum(c_sel + cnt_at, CAP)

    UC = 4

    def collect_body(i, carry):
        offg, offe, idxv = carry
        for u in range(UC):
            k = keys[pl.ds((i * UC + u) * L, L)]
            gt = k > t_key
            eq = k == t_key
            dest_g = offg + plsc.cumsum(gt.astype(jnp.int32)) - 1
            dest_e = offe + plsc.cumsum(eq.astype(jnp.int32)) - 1
            dest = jnp.where(gt, dest_g, dest_e)
            okm = jnp.logical_or(
                gt, jnp.logical_and(eq, dest_e < CAP))
            idx = idxv + u * L
            plsc.store_scatter(selk, [dest], k, mask=okm)
            plsc.store_scatter(seli, [dest], idx, mask=okm)
            offg = offg + plsc.all_reduce_population_count(gt)
            offe = offe + plsc.all_reduce_population_count(eq)
        return offg, offe, idxv + UC * L

    lax.fori_loop(0, NV // UC, collect_body, (zeros, zeros + c_sel, iota))

    nv_sel = CAP // L
    bufs = [(selk, seli), (selk2, seli2)]
    for p in range(6):
        srck, srci = bufs[p % 2]
        dstk, dsti = bufs[(p + 1) % 2]
        shift = 5 * p

        bins[pl.ds(0, L)] = zeros
        bins[pl.ds(L, L)] = zeros

        def count_body(i, _, srck=srck, shift=shift):
            pos = iota + i * L
            valid = pos < m
            k = srck[pl.ds(i * L, L)]
            dd = 31 - ((k >> shift) & 31)
            plsc.addupdate_scatter(bins, [dd], ones, mask=valid)
            return 0

        lax.fori_loop(0, nv_sel, count_body, 0)

        v0 = bins[pl.ds(0, L)]
        v1 = bins[pl.ds(L, L)]
        bins[pl.ds(0, L)] = plsc.cumsum(v0) - v0
        bins[pl.ds(L, L)] = plsc.cumsum(v1) - v1 + jnp.sum(v0)

        def perm_body(i, _, srck=srck, srci=srci, dstk=dstk, dsti=dsti,
                      shift=shift):
            pos = iota + i * L
            valid = pos < m
            k = srck[pl.ds(i * L, L)]
            iv = srci[pl.ds(i * L, L)]
            dd = 31 - ((k >> shift) & 31)
            occ, lm = plsc.scan_count(dd, mask=valid)
            base = plsc.load_gather(bins, [dd])
            dest = base + occ - 1
            plsc.store_scatter(dstk, [dest], k, mask=valid)
            plsc.store_scatter(dsti, [dest], iv, mask=valid)
            plsc.addupdate_scatter(bins, [dd], occ,
                                   mask=jnp.logical_and(lm, valid))
            return 0

        lax.fori_loop(0, nv_sel, perm_body, 0)

    def out_body(i, _):
        outv[pl.ds(i * L, L)] = plsc.bitcast(selk[pl.ds(i * L, L)], jnp.float32)
        return 0

    lax.fori_loop(0, TOPK // L, out_body, 0)

    pltpu.sync_copy(outv, vals_hbm.at[pl.ds(e * TOPK, TOPK)])
    pltpu.sync_copy(seli.at[pl.ds(0, TOPK)], idx_hbm.at[pl.ds(e * TOPK, TOPK)])


def _make_topk_sc():
    mesh = plsc.VectorSubcoreMesh(core_axis_name="c", subcore_axis_name="s")

    @functools.partial(
        pl.kernel,
        out_type=(
            jax.ShapeDtypeStruct((NUM_EXPERTS * TOPK,), jnp.float32),
            jax.ShapeDtypeStruct((NUM_EXPERTS * TOPK,), jnp.int32),
        ),
        mesh=mesh,
        compiler_params=pltpu.CompilerParams(needs_layout_passes=False),
        scratch_types=[
            pltpu.VMEM((N_TOKENS,), jnp.int32),
            pltpu.VMEM((N_TOKENS,), jnp.int32),
            pltpu.VMEM((CAP,), jnp.int32),
            pltpu.VMEM((CAP,), jnp.int32),
            pltpu.VMEM((CAP,), jnp.int32),
            pltpu.VMEM((CAP,), jnp.int32),
            pltpu.VMEM((2 * L,), jnp.int32),
            pltpu.VMEM((TOPK,), jnp.float32),
            pltpu.SMEM((8,), jnp.int32),
        ],
    )
    def topk_sc(scores_hbm, vals_hbm, idx_hbm,
                keys, hist, selk, seli, selk2, seli2, bins, outv, smem):
        wid = lax.axis_index("s") * 2 + lax.axis_index("c")
        for r in range(2):
            _topk_row(scores_hbm, vals_hbm, idx_hbm, wid * 2 + r,
                      keys, hist, selk, seli, selk2, seli2, bins, outv, smem)

    return topk_sc


_topk_sc = _make_topk_sc()


@jax.jit
def kernel(x, W_gate, b_gate):
    score_bits = _gate_scores(x, W_gate, b_gate)
    vals, idx = _topk_sc(score_bits.reshape(-1))
    return vals.reshape(NUM_EXPERTS, TOPK), idx.reshape(NUM_EXPERTS, TOPK)


# speedup vs baseline: 6.6113x; 1.5401x over previous
import functools

import jax
import jax.numpy as jnp
from jax import lax
from jax.experimental import pallas as pl
from jax.experimental.pallas import tpu as pltpu
from jax.experimental.pallas import tpu_sc as plsc

DIM = 768
NUM_EXPERTS = 64
N_TOKENS = 32768
TOPK = 512
BT = 2048

L = 16
NV = N_TOKENS // L


def _gate_body(x_ref, w_ref, b_ref, out_ref):
    xb = x_ref[...]
    w = w_ref[...]
    logits = lax.dot_general(
        w, xb, (((1,), (1,)), ((), ())),
        preferred_element_type=jnp.float32)
    logits = logits + b_ref[...][:, None]
    scores = jax.nn.sigmoid(logits)
    out_ref[...] = lax.bitcast_convert_type(scores, jnp.int32)


def _gate_scores(x, W_gate, b_gate):
    grid = (N_TOKENS // BT,)
    return pl.pallas_call(
        _gate_body,
        grid=grid,
        in_specs=[
            pl.BlockSpec((BT, DIM), lambda i: (i, 0)),
            pl.BlockSpec((NUM_EXPERTS, DIM), lambda i: (0, 0)),
            pl.BlockSpec((NUM_EXPERTS,), lambda i: (0,)),
        ],
        out_specs=pl.BlockSpec((NUM_EXPERTS, BT), lambda i: (0, i)),
        out_shape=jax.ShapeDtypeStruct((NUM_EXPERTS, N_TOKENS), jnp.int32),
    )(x, W_gate, b_gate)


def _iota16():
    return lax.broadcasted_iota(jnp.int32, (L,), 0)


def _find_threshold(hist, nvreg, target, smem, slot):
    iota = _iota16()

    def cond(state):
        _, carry = state
        return carry < target

    def body(state):
        j, carry = state
        v = hist[pl.ds(j * L, L)]
        s = jnp.sum(v)
        new = carry + s

        @pl.when(new >= target)
        def _():
            rv = lax.rev(v, (0,))
            dcum = lax.rev(plsc.cumsum(rv), (0,)) + carry
            cond_v = (dcum >= target).astype(jnp.int32)
            m = jnp.sum(cond_v)
            lane = m - 1
            sel = iota == lane
            zeros = jnp.zeros((L,), jnp.int32)
            smem[slot] = j * L + lane
            smem[slot + 1] = jnp.sum(jnp.where(sel, dcum - v, zeros))

        return j - 1, new

    lax.while_loop(cond, body, (jnp.int32(nvreg - 1), jnp.int32(0)))


def _topk_row(scores_hbm, vals_hbm, idx_hbm, e,
              keys, hist, selk, seli, selk2, seli2, bins, outv, smem):
    iota = _iota16()
    zeros = jnp.zeros((L,), jnp.int32)
    ones = jnp.ones((L,), jnp.int32)

    pltpu.sync_copy(scores_hbm.at[pl.ds(e * N_TOKENS, N_TOKENS)], keys)

    U = 8

    def clear_body(i, _):
        for u in range(U):
            hist[pl.ds((i * U + u) * L, L)] = zeros
        return 0

    lax.fori_loop(0, NV // U, clear_body, 0)

    def hist_hi_body(i, _):
        ks = [keys[pl.ds((i * U + u) * L, L)] for u in range(U)]
        bs = [k >> 15 for k in ks]
        for b in bs:
            plsc.addupdate_scatter(hist, [b], ones)
        return 0

    lax.fori_loop(0, NV // U, hist_hi_body, 0)

    _find_threshold(hist, NV, jnp.int32(TOPK), smem, 0)
    h_star = smem[0]
    c_gt = smem[1]

    lax.fori_loop(0, NV // U, clear_body, 0)

    def hist_lo_body(i, _):
        ks = [keys[pl.ds((i * U + u) * L, L)] for u in range(U)]
        els = [(k >> 15) == h_star for k in ks]
        lows = [k & 0x7FFF for k in ks]
        for lo, el in zip(lows, els):
            plsc.addupdate_scatter(hist, [lo], ones, mask=el)
        return 0

    lax.fori_loop(0, NV // U, hist_lo_body, 0)

    _find_threshold(hist, NV, TOPK - c_gt, smem, 3)
    l_star = smem[3]
    c_gt2 = smem[4]

    t_key = (h_star << 15) | l_star
    c_sel = c_gt + c_gt2

    UC = 8

    def collect_body(i, carry):
        offg, offe, idxv = carry
        ks = [keys[pl.ds((i * UC + u) * L, L)] for u in range(UC)]
        gts = [k > t_key for k in ks]
        eqs = [k == t_key for k in ks]
        prefs_g = [plsc.cumsum(gt.astype(jnp.int32)) for gt in gts]
        prefs_e = [plsc.cumsum(eq.astype(jnp.int32)) for eq in eqs]
        cnts_g = [plsc.all_reduce_population_count(gt) for gt in gts]
        cnts_e = [plsc.all_reduce_population_count(eq) for eq in eqs]
        for u in range(UC):
            dest_g = offg + prefs_g[u] - 1
            dest_e = offe + prefs_e[u] - 1
            dest = jnp.where(gts[u], dest_g, dest_e)
            okm = jnp.logical_or(
                gts[u], jnp.logical_and(eqs[u], dest_e < TOPK))
            idx = idxv + u * L
            plsc.store_scatter(selk, [dest], ks[u], mask=okm)
            plsc.store_scatter(seli, [dest], idx, mask=okm)
            offg = offg + cnts_g[u]
            offe = offe + cnts_e[u]
        return offg, offe, idxv + UC * L

    lax.fori_loop(0, NV // UC, collect_body, (zeros, zeros + c_sel, iota))

    nv_sel = TOPK // L
    bufs = [(selk, seli), (selk2, seli2)]
    for p in range(6):
        srck, srci = bufs[p % 2]
        dstk, dsti = bufs[(p + 1) % 2]
        shift = 5 * p

        bins[pl.ds(0, L)] = zeros
        bins[pl.ds(L, L)] = zeros

        UB = 8

        def count_body(i, _, srck=srck, shift=shift):
            ks = [srck[pl.ds((i * UB + u) * L, L)] for u in range(UB)]
            dds = [31 - ((k >> shift) & 31) for k in ks]
            for dd in dds:
                plsc.addupdate_scatter(bins, [dd], ones)
            return 0

        lax.fori_loop(0, nv_sel // UB, count_body, 0)

        v0 = bins[pl.ds(0, L)]
        v1 = bins[pl.ds(L, L)]
        bins[pl.ds(0, L)] = plsc.cumsum(v0) - v0
        bins[pl.ds(L, L)] = plsc.cumsum(v1) - v1 + jnp.sum(v0)

        UP = 4

        def perm_body(i, _, srck=srck, srci=srci, dstk=dstk, dsti=dsti,
                      shift=shift):
            ks = [srck[pl.ds((i * UP + u) * L, L)] for u in range(UP)]
            ivs = [srci[pl.ds((i * UP + u) * L, L)] for u in range(UP)]
            dds = [31 - ((k >> shift) & 31) for k in ks]
            scans = [plsc.scan_count(dd) for dd in dds]
            for u in range(UP):
                occ, lm = scans[u]
                base = plsc.load_gather(bins, [dds[u]])
                dest = base + occ - 1
                plsc.store_scatter(dstk, [dest], ks[u])
                plsc.store_scatter(dsti, [dest], ivs[u])
                plsc.addupdate_scatter(bins, [dds[u]], occ, mask=lm)
            return 0

        lax.fori_loop(0, nv_sel // UP, perm_body, 0)

    UO = 8

    def out_body(i, _):
        ks = [selk[pl.ds((i * UO + u) * L, L)] for u in range(UO)]
        vs = [plsc.bitcast(k, jnp.float32) for k in ks]
        for u in range(UO):
            outv[pl.ds((i * UO + u) * L, L)] = vs[u]
        return 0

    lax.fori_loop(0, TOPK // L // UO, out_body, 0)

    pltpu.sync_copy(outv, vals_hbm.at[pl.ds(e * TOPK, TOPK)])
    pltpu.sync_copy(seli.at[pl.ds(0, TOPK)], idx_hbm.at[pl.ds(e * TOPK, TOPK)])


def _make_topk_sc():
    mesh = plsc.VectorSubcoreMesh(core_axis_name="c", subcore_axis_name="s")

    @functools.partial(
        pl.kernel,
        out_type=(
            jax.ShapeDtypeStruct((NUM_EXPERTS * TOPK,), jnp.float32),
            jax.ShapeDtypeStruct((NUM_EXPERTS * TOPK,), jnp.int32),
        ),
        mesh=mesh,
        compiler_params=pltpu.CompilerParams(needs_layout_passes=False),
        scratch_types=[
            pltpu.VMEM((N_TOKENS,), jnp.int32),
            pltpu.VMEM((N_TOKENS,), jnp.int32),
            pltpu.VMEM((TOPK,), jnp.int32),
            pltpu.VMEM((TOPK,), jnp.int32),
            pltpu.VMEM((TOPK,), jnp.int32),
            pltpu.VMEM((TOPK,), jnp.int32),
            pltpu.VMEM((2 * L,), jnp.int32),
            pltpu.VMEM((TOPK,), jnp.float32),
            pltpu.SMEM((8,), jnp.int32),
        ],
    )
    def topk_sc(scores_hbm, vals_hbm, idx_hbm,
                keys, hist, selk, seli, selk2, seli2, bins, outv, smem):
        wid = lax.axis_index("s") * 2 + lax.axis_index("c")
        for r in range(2):
            _topk_row(scores_hbm, vals_hbm, idx_hbm, wid * 2 + r,
                      keys, hist, selk, seli, selk2, seli2, bins, outv, smem)

    return topk_sc


_topk_sc = _make_topk_sc()


@jax.jit
def kernel(x, W_gate, b_gate):
    score_bits = _gate_scores(x, W_gate, b_gate)
    vals, idx = _topk_sc(score_bits.reshape(-1))
    return vals.reshape(NUM_EXPERTS, TOPK), idx.reshape(NUM_EXPERTS, TOPK)


# speedup vs baseline: 9.4293x; 1.4262x over previous
import functools

import jax
import jax.numpy as jnp
from jax import lax
from jax.experimental import pallas as pl
from jax.experimental.pallas import tpu as pltpu
from jax.experimental.pallas import tpu_sc as plsc

DIM = 768
NUM_EXPERTS = 64
N_TOKENS = 32768
TOPK = 512
BT = 2048

L = 16
NV = N_TOKENS // L


def _gate_body(x_ref, w_ref, b_ref, out_ref):
    xb = x_ref[...]
    w = w_ref[...]
    logits = lax.dot_general(
        w, xb, (((1,), (1,)), ((), ())),
        preferred_element_type=jnp.float32)
    logits = logits + b_ref[...][:, None]
    scores = jax.nn.sigmoid(logits)
    out_ref[...] = lax.bitcast_convert_type(scores, jnp.int32)


def _gate_scores(x, W_gate, b_gate):
    grid = (N_TOKENS // BT,)
    return pl.pallas_call(
        _gate_body,
        grid=grid,
        in_specs=[
            pl.BlockSpec((BT, DIM), lambda i: (i, 0)),
            pl.BlockSpec((NUM_EXPERTS, DIM), lambda i: (0, 0)),
            pl.BlockSpec((NUM_EXPERTS,), lambda i: (0,)),
        ],
        out_specs=pl.BlockSpec((NUM_EXPERTS, BT), lambda i: (0, i)),
        out_shape=jax.ShapeDtypeStruct((NUM_EXPERTS, N_TOKENS), jnp.int32),
    )(x, W_gate, b_gate)


def _iota16():
    return lax.broadcasted_iota(jnp.int32, (L,), 0)


def _lane_cross(v, carry, target, iota):
    rv = lax.rev(v, (0,))
    dcum = lax.rev(plsc.cumsum(rv), (0,)) + carry
    cond_v = (dcum >= target).astype(jnp.int32)
    lane = jnp.sum(cond_v) - 1
    sel = iota == lane
    zeros = jnp.zeros((L,), jnp.int32)
    above = jnp.sum(jnp.where(sel, dcum - v, zeros))
    return lane, above


def _find_threshold(hist, coarse, ncoarse_v, target, smem, slot):
    iota = _iota16()

    def cond(state):
        _, carry = state
        return carry < target

    def body(state):
        j, carry = state
        v = coarse[pl.ds(j * L, L)]
        s = jnp.sum(v)
        new = carry + s

        @pl.when(new >= target)
        def _():
            lane, above = _lane_cross(v, carry, target, iota)
            smem[6] = j * L + lane
            smem[7] = above

        return j - 1, new

    lax.while_loop(cond, body, (jnp.int32(ncoarse_v - 1), jnp.int32(0)))

    cb = smem[6]
    carry2 = smem[7]
    v = hist[pl.ds(cb * L, L)]
    lane, above = _lane_cross(v, carry2, target, iota)
    smem[slot] = cb * L + lane
    smem[slot + 1] = above


def _topk_row(scores_hbm, vals_hbm, idx_hbm, e,
              keys, hist, coarse, selk, seli, selk2, seli2, bins, outv, smem):
    iota = _iota16()
    zeros = jnp.zeros((L,), jnp.int32)
    ones = jnp.ones((L,), jnp.int32)

    pltpu.sync_copy(scores_hbm.at[pl.ds(e * N_TOKENS, N_TOKENS)], keys)

    U = 8
    NCV = NV // L

    def clear_body(i, _):
        for u in range(U):
            hist[pl.ds((i * U + u) * L, L)] = zeros
        return 0

    def clear_coarse_body(i, _):
        for u in range(U):
            coarse[pl.ds((i * U + u) * L, L)] = zeros
        return 0

    lax.fori_loop(0, NV // U, clear_body, 0)
    lax.fori_loop(0, NCV // U, clear_coarse_body, 0)

    def hist_hi_body(i, _):
        ks = [keys[pl.ds((i * U + u) * L, L)] for u in range(U)]
        bs = [k >> 15 for k in ks]
        cbs = [k >> 19 for k in ks]
        for b, cb in zip(bs, cbs):
            plsc.addupdate_scatter(hist, [b], ones)
            plsc.addupdate_scatter(coarse, [cb], ones)
        return 0

    lax.fori_loop(0, NV // U, hist_hi_body, 0)

    _find_threshold(hist, coarse, NCV, jnp.int32(TOPK), smem, 0)
    h_star = smem[0]
    c_gt = smem[1]

    lax.fori_loop(0, NV // U, clear_body, 0)
    lax.fori_loop(0, NCV // U, clear_coarse_body, 0)

    def hist_lo_body(i, _):
        ks = [keys[pl.ds((i * U + u) * L, L)] for u in range(U)]
        els = [(k >> 15) == h_star for k in ks]
        lows = [k & 0x7FFF for k in ks]
        for lo, el in zip(lows, els):
            plsc.addupdate_scatter(hist, [lo], ones, mask=el)
            plsc.addupdate_scatter(coarse, [lo >> 4], ones, mask=el)
        return 0

    lax.fori_loop(0, NV // U, hist_lo_body, 0)

    _find_threshold(hist, coarse, NCV, TOPK - c_gt, smem, 3)
    l_star = smem[3]
    c_gt2 = smem[4]

    t_key = (h_star << 15) | l_star
    c_sel = c_gt + c_gt2

    UC = 8

    def gt_body(i, carry):
        offg, idxv = carry
        ks = [keys[pl.ds((i * UC + u) * L, L)] for u in range(UC)]
        gts = [k > t_key for k in ks]
        prefs = [plsc.cumsum(gt.astype(jnp.int32)) for gt in gts]
        cnts = [plsc.all_reduce_population_count(gt) for gt in gts]
        for u in range(UC):
            dest = offg + prefs[u] - 1
            plsc.store_scatter(selk, [dest], ks[u], mask=gts[u])
            plsc.store_scatter(seli, [dest], idxv + u * L, mask=gts[u])
            offg = offg + cnts[u]
        return offg, idxv + UC * L

    lax.fori_loop(0, NV // UC, gt_body, (zeros, iota))

    t_vec = zeros + t_key

    def tie_body(i, carry):
        offe, idxv = carry
        ks = [keys[pl.ds((i * UC + u) * L, L)] for u in range(UC)]
        eqs = [k == t_key for k in ks]
        prefs = [plsc.cumsum(eq.astype(jnp.int32)) for eq in eqs]
        cnts = [plsc.all_reduce_population_count(eq) for eq in eqs]
        for u in range(UC):
            dest = offe + prefs[u] - 1
            okm = jnp.logical_and(eqs[u], dest < TOPK)
            plsc.store_scatter(selk, [dest], t_vec, mask=okm)
            plsc.store_scatter(seli, [dest], idxv + u * L, mask=okm)
            offe = offe + cnts[u]
        return offe, idxv + UC * L

    lax.fori_loop(0, NV // UC, tie_body, (zeros + c_sel, iota))

    nv_sel = TOPK // L
    bufs = [(selk, seli), (selk2, seli2)]
    for p in range(6):
        srck, srci = bufs[p % 2]
        dstk, dsti = bufs[(p + 1) % 2]
        shift = 5 * p

        bins[pl.ds(0, L)] = zeros
        bins[pl.ds(L, L)] = zeros

        UB = 8

        def count_body(i, _, srck=srck, shift=shift):
            ks = [srck[pl.ds((i * UB + u) * L, L)] for u in range(UB)]
            dds = [31 - ((k >> shift) & 31) for k in ks]
            for dd in dds:
                plsc.addupdate_scatter(bins, [dd], ones)
            return 0

        lax.fori_loop(0, nv_sel // UB, count_body, 0)

        v0 = bins[pl.ds(0, L)]
        v1 = bins[pl.ds(L, L)]
        bins[pl.ds(0, L)] = plsc.cumsum(v0) - v0
        bins[pl.ds(L, L)] = plsc.cumsum(v1) - v1 + jnp.sum(v0)

        UP = 4

        def perm_body(i, _, srck=srck, srci=srci, dstk=dstk, dsti=dsti,
                      shift=shift):
            ks = [srck[pl.ds((i * UP + u) * L, L)] for u in range(UP)]
            ivs = [srci[pl.ds((i * UP + u) * L, L)] for u in range(UP)]
            dds = [31 - ((k >> shift) & 31) for k in ks]
            scans = [plsc.scan_count(dd) for dd in dds]
            for u in range(UP):
                occ, lm = scans[u]
                base = plsc.load_gather(bins, [dds[u]])
                dest = base + occ - 1
                plsc.store_scatter(dstk, [dest], ks[u])
                plsc.store_scatter(dsti, [dest], ivs[u])
                plsc.addupdate_scatter(bins, [dds[u]], occ, mask=lm)
            return 0

        lax.fori_loop(0, nv_sel // UP, perm_body, 0)

    UO = 8

    def out_body(i, _):
        ks = [selk[pl.ds((i * UO + u) * L, L)] for u in range(UO)]
        vs = [plsc.bitcast(k, jnp.float32) for k in ks]
        for u in range(UO):
            outv[pl.ds((i * UO + u) * L, L)] = vs[u]
        return 0

    lax.fori_loop(0, TOPK // L // UO, out_body, 0)

    pltpu.sync_copy(outv, vals_hbm.at[pl.ds(e * TOPK, TOPK)])
    pltpu.sync_copy(seli.at[pl.ds(0, TOPK)], idx_hbm.at[pl.ds(e * TOPK, TOPK)])


def _make_topk_sc():
    mesh = plsc.VectorSubcoreMesh(core_axis_name="c", subcore_axis_name="s")

    @functools.partial(
        pl.kernel,
        out_type=(
            jax.ShapeDtypeStruct((NUM_EXPERTS * TOPK,), jnp.float32),
            jax.ShapeDtypeStruct((NUM_EXPERTS * TOPK,), jnp.int32),
        ),
        mesh=mesh,
        compiler_params=pltpu.CompilerParams(needs_layout_passes=False),
        scratch_types=[
            pltpu.VMEM((N_TOKENS,), jnp.int32),
            pltpu.VMEM((N_TOKENS,), jnp.int32),
            pltpu.VMEM((N_TOKENS // L,), jnp.int32),
            pltpu.VMEM((TOPK,), jnp.int32),
            pltpu.VMEM((TOPK,), jnp.int32),
            pltpu.VMEM((TOPK,), jnp.int32),
            pltpu.VMEM((TOPK,), jnp.int32),
            pltpu.VMEM((2 * L,), jnp.int32),
            pltpu.VMEM((TOPK,), jnp.float32),
            pltpu.SMEM((8,), jnp.int32),
        ],
    )
    def topk_sc(scores_hbm, vals_hbm, idx_hbm,
                keys, hist, coarse, selk, seli, selk2, seli2, bins, outv,
                smem):
        wid = lax.axis_index("s") * 2 + lax.axis_index("c")
        for r in range(2):
            _topk_row(scores_hbm, vals_hbm, idx_hbm, wid * 2 + r,
                      keys, hist, coarse, selk, seli, selk2, seli2, bins,
                      outv, smem)

    return topk_sc


_topk_sc = _make_topk_sc()


@jax.jit
def kernel(x, W_gate, b_gate):
    score_bits = _gate_scores(x, W_gate, b_gate)
    vals, idx = _topk_sc(score_bits.reshape(-1))
    return vals.reshape(NUM_EXPERTS, TOPK), idx.reshape(NUM_EXPERTS, TOPK)
